# speedup vs baseline: 1.8308x; 1.8308x over previous
import functools

import jax
import jax.numpy as jnp
from jax import lax
from jax.experimental import pallas as pl
from jax.experimental.pallas import tpu as pltpu
from jax.experimental.pallas import tpu_sc as plsc

VOCAB = 100000
D = 128
L = 200
B = 1024
NC = 2
NS = 16
NW = NC * NS
TOK = B * L
PER_W = TOK // NW
K = 128
NCH = PER_W // K

_mesh = plsc.VectorSubcoreMesh(
    core_axis_name="c", subcore_axis_name="s", num_cores=NC, num_subcores=NS
)


@functools.partial(
    pl.kernel,
    out_type=jax.ShapeDtypeStruct((TOK, D), jnp.float32),
    mesh=_mesh,
    scratch_types=[
        pltpu.VMEM((K,), jnp.int32),
        pltpu.VMEM((K, D), jnp.float32),
        pltpu.VMEM((L, D), jnp.float32),
        pltpu.SemaphoreType.DMA,
    ],
)
def _embed_sc(x_hbm, tok_hbm, pos_hbm, out_hbm, idx_v, rows_v, pos_v, sem):
    w = lax.axis_index("s") * NC + lax.axis_index("c")
    base = w * PER_W
    pltpu.sync_copy(pos_hbm.at[pl.ds(0, L)], pos_v)

    def chunk(c, carry):
        off = base + c * K
        pltpu.sync_copy(x_hbm.at[pl.ds(off, K)], idx_v)
        pltpu.async_copy(tok_hbm.at[idx_v], rows_v, sem).wait()
        l0 = lax.rem(c * K, L)

        def tok_body(t, carry2):
            l = lax.rem(l0 + t, L)
            for j in range(D // 16):
                s = pl.ds(j * 16, 16)
                rows_v[t, s] = rows_v[t, s] + pos_v[l, s]
            return carry2

        lax.fori_loop(0, K, tok_body, 0)
        pltpu.sync_copy(rows_v, out_hbm.at[pl.ds(off, K)])
        return carry

    lax.fori_loop(0, NCH, chunk, 0)


def kernel(x, token_table, pos_table):
    xf = x.reshape(-1).astype(jnp.int32)
    out = _embed_sc(xf, token_table, pos_table)
    return out.reshape(B, L, D)
